# baseline (speedup 1.0000x reference)
import functools

import jax
import jax.numpy as jnp
from jax import lax
from jax.experimental import pallas as pl
from jax.experimental.pallas import tpu as pltpu
from jax.experimental.pallas import tpu_sc as plsc

NUM_CLASSES = 1000
BATCH = 16384
LANES = 16
NUM_WORKERS = 16
ROWS_PER_W = BATCH // NUM_WORKERS
CHUNK_ROWS = 32
CHUNK_PAIRS = ROWS_PER_W // (2 * CHUNK_ROWS)

_mesh = plsc.VectorSubcoreMesh(core_axis_name="c", subcore_axis_name="s", num_cores=1)


@functools.partial(
    pl.kernel,
    out_type=jax.ShapeDtypeStruct((BATCH, NUM_CLASSES), jnp.float32),
    mesh=_mesh,
    scratch_types=[
        pltpu.VMEM((CHUNK_ROWS, NUM_CLASSES), jnp.float32),
        pltpu.VMEM((CHUNK_ROWS, NUM_CLASSES), jnp.float32),
        pltpu.VMEM((ROWS_PER_W,), jnp.int32),
        pltpu.SemaphoreType.DMA,
        pltpu.SemaphoreType.DMA,
    ],
)
def _onehot_body(labels_hbm, out_hbm, tmpl_a, tmpl_b, lab_v, sem_a, sem_b):
    cid = lax.axis_index("c")
    sid = lax.axis_index("s")
    wid = sid + cid * 16
    row0 = wid * ROWS_PER_W

    pltpu.sync_copy(labels_hbm.at[pl.ds(wid * ROWS_PER_W, ROWS_PER_W)], lab_v)

    minus_ten = jnp.full((LANES,), -10.0, jnp.float32)
    iota16 = lax.iota(jnp.int32, LANES)
    col_groups = [k * LANES for k in range(NUM_CLASSES // LANES)] + [NUM_CLASSES - LANES]

    def fill(tmpl):
        def fill_row(r, carry):
            for c0 in col_groups:
                tmpl[r, pl.ds(c0, LANES)] = minus_ten
            return carry
        lax.fori_loop(0, CHUNK_ROWS, fill_row, 0)

    fill(tmpl_a)
    fill(tmpl_b)

    def place(c, tmpl):
        for g in range(CHUNK_ROWS // LANES):
            lab16 = lab_v[pl.ds(c * CHUNK_ROWS + g * LANES, LANES)]
            for e in range(LANES):
                lab = lab16[e]
                val = jnp.where(iota16 == (lab & (LANES - 1)), 10.0, -10.0
                                ).astype(jnp.float32)
                col0 = pl.multiple_of((lab >> 4) << 4, LANES)
                tmpl[g * LANES + e, pl.ds(col0, LANES)] = val

    def restore(c, tmpl):
        for g in range(CHUNK_ROWS // LANES):
            lab16 = lab_v[pl.ds(c * CHUNK_ROWS + g * LANES, LANES)]
            for e in range(LANES):
                lab = lab16[e]
                col0 = pl.multiple_of((lab >> 4) << 4, LANES)
                tmpl[g * LANES + e, pl.ds(col0, LANES)] = minus_ten

    def pair_body(i, carry):
        for slot, (tmpl, sem) in enumerate(((tmpl_a, sem_a), (tmpl_b, sem_b))):
            c = 2 * i + slot

            @pl.when(i > 0)
            def _wait_and_restore():
                pltpu.make_async_copy(
                    tmpl, out_hbm.at[pl.ds(0, CHUNK_ROWS)], sem).wait()
                restore(c - 2, tmpl)

            place(c, tmpl)
            pltpu.async_copy(
                tmpl, out_hbm.at[pl.ds(row0 + c * CHUNK_ROWS, CHUNK_ROWS)], sem)
        return carry

    lax.fori_loop(0, CHUNK_PAIRS, pair_body, 0)

    pltpu.make_async_copy(tmpl_a, out_hbm.at[pl.ds(0, CHUNK_ROWS)], sem_a).wait()
    pltpu.make_async_copy(tmpl_b, out_hbm.at[pl.ds(0, CHUNK_ROWS)], sem_b).wait()


def kernel(input_ids, dummy):
    labels = input_ids[:, 0].astype(jnp.int32)
    return _onehot_body(labels)


# speedup vs baseline: 2.4166x; 2.4166x over previous
import functools

import jax
import jax.numpy as jnp
from jax import lax
from jax.experimental import pallas as pl
from jax.experimental.pallas import tpu as pltpu
from jax.experimental.pallas import tpu_sc as plsc

NUM_CLASSES = 1000
BATCH = 16384
LANES = 16
NUM_WORKERS = 32
COLS_PER_W = BATCH // NUM_WORKERS
CHUNK_COLS = 128
NCHUNKS = COLS_PER_W // CHUNK_COLS

_mesh = plsc.VectorSubcoreMesh(core_axis_name="c", subcore_axis_name="s")


@functools.partial(
    pl.kernel,
    out_type=jax.ShapeDtypeStruct((NUM_CLASSES, BATCH), jnp.float32),
    mesh=_mesh,
    scratch_types=[
        pltpu.VMEM((NUM_CLASSES, CHUNK_COLS), jnp.float32),
        pltpu.VMEM((COLS_PER_W,), jnp.int32),
        pltpu.SemaphoreType.DMA,
    ],
)
def _onehot_body(labels_hbm, out_hbm, tmpl, lab_v, sem):
    cid = lax.axis_index("c")
    sid = lax.axis_index("s")
    wid = sid * 2 + cid
    col0 = wid * COLS_PER_W

    pltpu.sync_copy(labels_hbm.at[pl.ds(col0, COLS_PER_W)], lab_v)

    minus_ten = jnp.full((LANES,), -10.0, jnp.float32)
    iota16 = lax.iota(jnp.int32, LANES)

    def fill_row(r, carry):
        for g in range(CHUNK_COLS // LANES):
            tmpl[r, pl.ds(g * LANES, LANES)] = minus_ten
        return carry
    lax.fori_loop(0, NUM_CLASSES, fill_row, 0, unroll=4)

    def place(c):
        for g in range(CHUNK_COLS // LANES):
            lab16 = lab_v[pl.ds(c * CHUNK_COLS + g * LANES, LANES)]
            for e in range(LANES):
                lab = lab16[e]
                old = tmpl[lab, pl.ds(g * LANES, LANES)]
                tmpl[lab, pl.ds(g * LANES, LANES)] = jnp.where(
                    iota16 == e, jnp.float32(10.0), old)

    def restore(c):
        for g in range(CHUNK_COLS // LANES):
            lab16 = lab_v[pl.ds(c * CHUNK_COLS + g * LANES, LANES)]
            for e in range(LANES):
                lab = lab16[e]
                tmpl[lab, pl.ds(g * LANES, LANES)] = minus_ten

    def chunk_body(c, carry):
        @pl.when(c > 0)
        def _wait_and_restore():
            pltpu.make_async_copy(
                tmpl, out_hbm.at[:, pl.ds(0, CHUNK_COLS)], sem).wait()
            restore(c - 1)

        place(c)
        start = pl.multiple_of(col0 + c * CHUNK_COLS, CHUNK_COLS)
        pltpu.async_copy(tmpl, out_hbm.at[:, pl.ds(start, CHUNK_COLS)], sem)
        return carry

    lax.fori_loop(0, NCHUNKS, chunk_body, 0)

    pltpu.make_async_copy(
        tmpl, out_hbm.at[:, pl.ds(0, CHUNK_COLS)], sem).wait()


def kernel(input_ids, dummy):
    labels = input_ids[:, 0].astype(jnp.int32)
    return _onehot_body(labels).T


# speedup vs baseline: 2.4921x; 1.0312x over previous
import functools

import jax
import jax.numpy as jnp
from jax import lax
from jax.experimental import pallas as pl
from jax.experimental.pallas import tpu as pltpu
from jax.experimental.pallas import tpu_sc as plsc

NUM_CLASSES = 1000
BATCH = 16384
LANES = 16
NUM_WORKERS = 32
COLS_PER_W = BATCH // NUM_WORKERS
CHUNK_COLS = 128
NCHUNKS = COLS_PER_W // CHUNK_COLS

_mesh = plsc.VectorSubcoreMesh(core_axis_name="c", subcore_axis_name="s")


@functools.partial(
    pl.kernel,
    out_type=jax.ShapeDtypeStruct((NUM_CLASSES, BATCH), jnp.float32),
    mesh=_mesh,
    scratch_types=[
        pltpu.VMEM((NUM_CLASSES, CHUNK_COLS), jnp.float32),
        pltpu.VMEM((COLS_PER_W,), jnp.int32),
        pltpu.SemaphoreType.DMA,
        pltpu.SemaphoreType.DMA,
    ],
)
def _onehot_body(labels_hbm, out_hbm, tmpl, lab_v, sem, lab_sem):
    cid = lax.axis_index("c")
    sid = lax.axis_index("s")
    wid = sid * 2 + cid
    col0 = wid * COLS_PER_W

    pltpu.async_copy(labels_hbm.at[pl.ds(col0, COLS_PER_W)], lab_v, lab_sem)

    minus_ten = jnp.full((LANES,), -10.0, jnp.float32)
    iota16 = lax.iota(jnp.int32, LANES)

    def fill_row(r, carry):
        for g in range(CHUNK_COLS // LANES):
            tmpl[r, pl.ds(g * LANES, LANES)] = minus_ten
        return carry
    lax.fori_loop(0, NUM_CLASSES, fill_row, 0, unroll=8)

    pltpu.make_async_copy(
        labels_hbm.at[pl.ds(col0, COLS_PER_W)], lab_v, lab_sem).wait()

    def place(c):
        def group(g, carry):
            goff = pl.multiple_of(g * LANES, LANES)
            lab16 = lab_v[pl.ds(c * CHUNK_COLS + goff, LANES)]
            for e in range(LANES):
                lab = lab16[e]
                old = tmpl[lab, pl.ds(goff, LANES)]
                tmpl[lab, pl.ds(goff, LANES)] = jnp.where(
                    iota16 == e, jnp.float32(10.0), old)
            return carry
        lax.fori_loop(0, CHUNK_COLS // LANES, group, 0)

    def restore(c):
        def group(g, carry):
            goff = pl.multiple_of(g * LANES, LANES)
            lab16 = lab_v[pl.ds(c * CHUNK_COLS + goff, LANES)]
            for e in range(LANES):
                lab = lab16[e]
                tmpl[lab, pl.ds(goff, LANES)] = minus_ten
            return carry
        lax.fori_loop(0, CHUNK_COLS // LANES, group, 0)

    def chunk_body(c, carry):
        @pl.when(c > 0)
        def _wait_and_restore():
            pltpu.make_async_copy(
                tmpl, out_hbm.at[:, pl.ds(0, CHUNK_COLS)], sem).wait()
            restore(c - 1)

        place(c)
        start = pl.multiple_of(col0 + c * CHUNK_COLS, CHUNK_COLS)
        pltpu.async_copy(tmpl, out_hbm.at[:, pl.ds(start, CHUNK_COLS)], sem)
        return carry

    lax.fori_loop(0, NCHUNKS, chunk_body, 0)

    pltpu.make_async_copy(
        tmpl, out_hbm.at[:, pl.ds(0, CHUNK_COLS)], sem).wait()


def kernel(input_ids, dummy):
    labels = input_ids[:, 0].astype(jnp.int32)
    return _onehot_body(labels).T
